# Initial kernel scaffold; baseline (speedup 1.0000x reference)
import jax
import jax.numpy as jnp
from jax.experimental import pallas as pl


def kernel(node_features, node_attrs, edge_index, edge_embedding, edge_attrs, W1_0, Wfc1_0, Wfc2_0, We_0, W2_0, Wsc_0, bias_0, W1_1, Wfc1_1, Wfc2_1, We_1, W2_1, Wsc_1, bias_1):
    raise NotImplementedError("write your pallas kernel here")


# speedup vs baseline: 2.3605x; 2.3605x over previous
import functools
import math

import jax
import jax.numpy as jnp
from jax import lax
from jax.experimental import pallas as pl
from jax.experimental.pallas import tpu as pltpu
from jax.experimental.pallas import tpu_sc as plsc

N = 10000
E = 320000
D = 128
NS = 32
LOG2 = math.log(2.0)
CN = 1.0 / math.sqrt(E / N)

NSC = 2
NSUB = 16
NW = NSC * NSUB
EPW = E // NW
CHUNK = 80
NCHUNK = EPW // CHUNK
NPAD = 10240
RPW = NPAD // NSUB

BN = 1000
BE = 4000


def _node_prep_body(feat_ref, attr_ref, w1_ref, wsc_ref, x_ref, sc_ref):
    f = feat_ref[...]
    x_ref[...] = jnp.dot(f, w1_ref[...], preferred_element_type=jnp.float32)
    sc_ref[...] = f * jnp.dot(attr_ref[...], wsc_ref[...],
                              preferred_element_type=jnp.float32)


def _node_prep(feat, attrs, w1, wsc):
    return pl.pallas_call(
        _node_prep_body,
        grid=(N // BN,),
        in_specs=[
            pl.BlockSpec((BN, D), lambda i: (i, 0)),
            pl.BlockSpec((BN, 16), lambda i: (i, 0)),
            pl.BlockSpec((D, D), lambda i: (0, 0)),
            pl.BlockSpec((16, D), lambda i: (0, 0)),
        ],
        out_specs=[
            pl.BlockSpec((BN, D), lambda i: (i, 0)),
            pl.BlockSpec((BN, D), lambda i: (i, 0)),
        ],
        out_shape=[
            jax.ShapeDtypeStruct((N, D), jnp.float32),
            jax.ShapeDtypeStruct((N, D), jnp.float32),
        ],
    )(feat, attrs, w1, wsc)


def _edge_we_body(ee_ref, ea_ref, wfc1_ref, wfc2_ref, wem_ref, out_ref):
    h = jnp.dot(ee_ref[...], wfc1_ref[...], preferred_element_type=jnp.float32)
    h = h * jax.nn.sigmoid(h)
    w = jnp.dot(h, wfc2_ref[...], preferred_element_type=jnp.float32)
    m = jnp.dot(ea_ref[...], wem_ref[...], preferred_element_type=jnp.float32)
    out_ref[...] = w * m


def _edge_we(ee, ea, wfc1, wfc2, wem):
    return pl.pallas_call(
        _edge_we_body,
        grid=(E // BE,),
        in_specs=[
            pl.BlockSpec((BE, 8), lambda i: (i, 0)),
            pl.BlockSpec((BE, 4), lambda i: (i, 0)),
            pl.BlockSpec((8, 8), lambda i: (0, 0)),
            pl.BlockSpec((8, D), lambda i: (0, 0)),
            pl.BlockSpec((4, D), lambda i: (0, 0)),
        ],
        out_specs=pl.BlockSpec((BE, D), lambda i: (i, 0)),
        out_shape=jax.ShapeDtypeStruct((E, D), jnp.float32),
    )(ee, ea, wfc1, wfc2, wem)


def _sc_agg_build():
    mesh = plsc.VectorSubcoreMesh(core_axis_name="c", subcore_axis_name="s")

    @functools.partial(
        pl.kernel,
        mesh=mesh,
        out_type=jax.ShapeDtypeStruct((NSC, NPAD, D), jnp.float32),
        scratch_types=[
            pltpu.VMEM((CHUNK,), jnp.int32),
            pltpu.VMEM((CHUNK,), jnp.int32),
            pltpu.VMEM((CHUNK, D), jnp.float32),
            pltpu.VMEM((CHUNK, D), jnp.float32),
            pltpu.VMEM_SHARED((NPAD, D), jnp.float32),
            pltpu.SemaphoreType.DMA,
        ],
    )
    def sc_agg(x_hbm, we_hbm, src_hbm, dst_hbm, zeros_hbm, out_hbm,
               src_v, dst_v, rows_v, we_v, acc_sh, sem):
        cid = lax.axis_index("c")
        sid = lax.axis_index("s")
        wid = sid * NSC + cid
        r0 = sid * RPW
        pltpu.sync_copy(zeros_hbm, acc_sh.at[pl.ds(r0, RPW)])
        plsc.subcore_barrier()
        base0 = wid * EPW

        def chunk_body(ci, carry):
            base = base0 + ci * CHUNK
            pltpu.sync_copy(src_hbm.at[pl.ds(base, CHUNK)], src_v)
            pltpu.sync_copy(dst_hbm.at[pl.ds(base, CHUNK)], dst_v)
            pltpu.async_copy(x_hbm.at[src_v], rows_v, sem).wait()
            pltpu.sync_copy(we_hbm.at[pl.ds(base, CHUNK)], we_v)

            def edge_body(e, c2):
                for j in range(D // 16):
                    sl = pl.ds(j * 16, 16)
                    rows_v[e, sl] = rows_v[e, sl] * we_v[e, sl]
                return c2

            lax.fori_loop(0, CHUNK, edge_body, 0)
            pltpu.sync_copy(rows_v, acc_sh.at[dst_v], add=True)
            return carry

        lax.fori_loop(0, NCHUNK, chunk_body, 0)
        plsc.subcore_barrier()
        pltpu.sync_copy(acc_sh.at[pl.ds(r0, RPW)],
                        out_hbm.at[cid, pl.ds(r0, RPW)])

    return sc_agg


@functools.cache
def _sc_agg_cached():
    return _sc_agg_build()


def _sc_agg_call(x, we, src, dst, zeros_rpw):
    return _sc_agg_cached()(x, we, src, dst, zeros_rpw)


def _finish_body(p_ref, sc_ref, w2_ref, g_ref, brow_ref, out_ref):
    agg = (p_ref[0] + p_ref[1]) * CN
    y = jnp.dot(agg, w2_ref[...], preferred_element_type=jnp.float32) + sc_ref[...]
    n2 = jnp.dot(y * y, g_ref[...], preferred_element_type=jnp.float32) + 1e-8
    nrm = jnp.sqrt(n2)
    t = nrm + brow_ref[...]
    sp = jnp.maximum(t, 0.0) + jnp.log1p(jnp.exp(-jnp.abs(t))) - LOG2
    out_ref[...] = y * (sp / nrm)


def _finish(parts, sc, w2, g, brow):
    return pl.pallas_call(
        _finish_body,
        grid=(N // BN,),
        in_specs=[
            pl.BlockSpec((NSC, BN, D), lambda i: (0, i, 0)),
            pl.BlockSpec((BN, D), lambda i: (i, 0)),
            pl.BlockSpec((D, D), lambda i: (0, 0)),
            pl.BlockSpec((D, D), lambda i: (0, 0)),
            pl.BlockSpec((1, D), lambda i: (0, 0)),
        ],
        out_specs=pl.BlockSpec((BN, D), lambda i: (i, 0)),
        out_shape=jax.ShapeDtypeStruct((N, D), jnp.float32),
    )(parts, sc, w2, g, brow)


def _norm_groups():
    p = jnp.arange(D)[:, None]
    q = jnp.arange(D)[None, :]
    scal = (p == q) & (q < NS)
    vec = (p >= NS) & (q >= NS) & ((p - NS) // 3 == (q - NS) // 3)
    return (scal | vec).astype(jnp.float32)


def _layer(feat, attrs, src, dst, ee, ea, w1, wfc1, wfc2, wem, w2, wsc, bias,
           zeros_rpw, g):
    x, sc = _node_prep(feat, attrs, w1, wsc)
    we = _edge_we(ee, ea, wfc1, wfc2, wem)
    parts = _sc_agg_call(x, we, src, dst, zeros_rpw)
    brow = jnp.where(jnp.arange(D) < NS, bias[0], bias[1])[None, :]
    return _finish(parts, sc, w2, g, brow)


def kernel(node_features, node_attrs, edge_index, edge_embedding, edge_attrs,
           W1_0, Wfc1_0, Wfc2_0, We_0, W2_0, Wsc_0, bias_0,
           W1_1, Wfc1_1, Wfc2_1, We_1, W2_1, Wsc_1, bias_1):
    src = edge_index[0].astype(jnp.int32)
    dst = edge_index[1].astype(jnp.int32)
    zeros_rpw = jnp.zeros((RPW, D), dtype=jnp.float32)
    g = _norm_groups()
    h = _layer(node_features, node_attrs, src, dst, edge_embedding, edge_attrs,
               W1_0, Wfc1_0, Wfc2_0, We_0, W2_0, Wsc_0, bias_0, zeros_rpw, g)
    h = _layer(h, node_attrs, src, dst, edge_embedding, edge_attrs,
               W1_1, Wfc1_1, Wfc2_1, We_1, W2_1, Wsc_1, bias_1, zeros_rpw, g)
    return h


# speedup vs baseline: 3.9518x; 1.6741x over previous
import functools
import math

import jax
import jax.numpy as jnp
from jax import lax
from jax.experimental import pallas as pl
from jax.experimental.pallas import tpu as pltpu
from jax.experimental.pallas import tpu_sc as plsc

N = 10000
E = 320000
D = 128
NS = 32
LOG2 = math.log(2.0)
CN = 1.0 / math.sqrt(E / N)

NSC = 2
NSUB = 16
NW = NSC * NSUB
EPW = E // NW
CHUNK = 40
NCHUNK = EPW // CHUNK
GRP = 10
NGRP = NCHUNK // GRP
IRING = 3
NPAD = 10240
RPW = NPAD // NSUB

BN = 1000
BE = 4000


def _node_prep_body(feat_ref, attr_ref, w1_ref, wsc_ref, x_ref, sc_ref):
    f = feat_ref[...]
    x_ref[...] = jnp.dot(f, w1_ref[...], preferred_element_type=jnp.float32)
    sc_ref[...] = f * jnp.dot(attr_ref[...], wsc_ref[...],
                              preferred_element_type=jnp.float32)


def _node_prep(feat, attrs, w1, wsc):
    return pl.pallas_call(
        _node_prep_body,
        grid=(N // BN,),
        in_specs=[
            pl.BlockSpec((BN, D), lambda i: (i, 0)),
            pl.BlockSpec((BN, 16), lambda i: (i, 0)),
            pl.BlockSpec((D, D), lambda i: (0, 0)),
            pl.BlockSpec((16, D), lambda i: (0, 0)),
        ],
        out_specs=[
            pl.BlockSpec((BN, D), lambda i: (i, 0)),
            pl.BlockSpec((BN, D), lambda i: (i, 0)),
        ],
        out_shape=[
            jax.ShapeDtypeStruct((N, D), jnp.float32),
            jax.ShapeDtypeStruct((N, D), jnp.float32),
        ],
    )(feat, attrs, w1, wsc)


def _edge_we_body(ee_ref, ea_ref, wfc1_ref, wfc2_ref, wem_ref, out_ref):
    h = jnp.dot(ee_ref[...], wfc1_ref[...], preferred_element_type=jnp.float32)
    h = h * jax.nn.sigmoid(h)
    w = jnp.dot(h, wfc2_ref[...], preferred_element_type=jnp.float32)
    m = jnp.dot(ea_ref[...], wem_ref[...], preferred_element_type=jnp.float32)
    out_ref[...] = w * m


def _edge_we(ee, ea, wfc1, wfc2, wem):
    return pl.pallas_call(
        _edge_we_body,
        grid=(E // BE,),
        in_specs=[
            pl.BlockSpec((BE, 8), lambda i: (i, 0)),
            pl.BlockSpec((BE, 4), lambda i: (i, 0)),
            pl.BlockSpec((8, 8), lambda i: (0, 0)),
            pl.BlockSpec((8, D), lambda i: (0, 0)),
            pl.BlockSpec((4, D), lambda i: (0, 0)),
        ],
        out_specs=pl.BlockSpec((BE, D), lambda i: (i, 0)),
        out_shape=jax.ShapeDtypeStruct((E, D), jnp.float32),
    )(ee, ea, wfc1, wfc2, wem)


def _sc_agg_build():
    mesh = plsc.VectorSubcoreMesh(core_axis_name="c", subcore_axis_name="s")

    @functools.partial(
        pl.kernel,
        mesh=mesh,
        out_type=jax.ShapeDtypeStruct((NSC, NPAD, D), jnp.float32),
        scratch_types=[
            pltpu.VMEM((IRING * GRP, 1, CHUNK), jnp.int32),
            pltpu.VMEM((IRING * GRP, 1, CHUNK), jnp.int32),
            pltpu.VMEM((2, CHUNK, D), jnp.float32),
            pltpu.VMEM((2, CHUNK, D), jnp.float32),
            pltpu.VMEM((2, CHUNK, D), jnp.float32),
            pltpu.VMEM_SHARED((NPAD, D), jnp.float32),
            pltpu.SemaphoreType.DMA,
            pltpu.SemaphoreType.DMA,
            pltpu.SemaphoreType.DMA,
            pltpu.SemaphoreType.DMA,
            pltpu.SemaphoreType.DMA,
            pltpu.SemaphoreType.DMA,
            pltpu.SemaphoreType.DMA,
            pltpu.SemaphoreType.DMA,
        ],
    )
    def sc_agg(x_hbm, we_hbm, src3_hbm, dst3_hbm, zeros_hbm, out_hbm,
               srcv, dstv, rows, webuf, msg, acc_sh,
               sg0, sg1, sw0, sw1, ss0, ss1, six, diy):
        cid = lax.axis_index("c")
        sid = lax.axis_index("s")
        wid = sid * NSC + cid
        r0 = sid * RPW
        sgs, sws, sss = (sg0, sg1), (sw0, sw1), (ss0, ss1)
        pltpu.sync_copy(zeros_hbm, acc_sh.at[pl.ds(r0, RPW)])
        c0 = wid * NCHUNK
        ebase0 = wid * EPW

        def islot(ci):
            return lax.rem(ci // GRP, IRING) * GRP + lax.rem(ci, GRP)

        def idx_issue(g):
            pltpu.async_copy(src3_hbm.at[pl.ds(c0 + g * GRP, GRP)],
                             srcv.at[pl.ds(lax.rem(g, IRING) * GRP, GRP)], six)
            pltpu.async_copy(dst3_hbm.at[pl.ds(c0 + g * GRP, GRP)],
                             dstv.at[pl.ds(lax.rem(g, IRING) * GRP, GRP)], diy)

        def idx_wait(g):
            pltpu.make_async_copy(
                src3_hbm.at[pl.ds(c0 + g * GRP, GRP)],
                srcv.at[pl.ds(lax.rem(g, IRING) * GRP, GRP)], six).wait()
            pltpu.make_async_copy(
                dst3_hbm.at[pl.ds(c0 + g * GRP, GRP)],
                dstv.at[pl.ds(lax.rem(g, IRING) * GRP, GRP)], diy).wait()

        for g0 in range(IRING):
            pltpu.sync_copy(src3_hbm.at[pl.ds(c0 + g0 * GRP, GRP)],
                            srcv.at[pl.ds(g0 * GRP, GRP)])
            pltpu.sync_copy(dst3_hbm.at[pl.ds(c0 + g0 * GRP, GRP)],
                            dstv.at[pl.ds(g0 * GRP, GRP)])
        plsc.subcore_barrier()

        def issue(ci, b):
            pltpu.async_copy(x_hbm.at[srcv.at[islot(ci), 0]], rows.at[b],
                             sgs[b])
            pltpu.async_copy(we_hbm.at[pl.ds(ebase0 + ci * CHUNK, CHUNK)],
                             webuf.at[b], sws[b])

        def step(ci, b, first):
            pltpu.make_async_copy(x_hbm.at[srcv.at[islot(ci), 0]], rows.at[b],
                                  sgs[b]).wait()
            pltpu.make_async_copy(we_hbm.at[pl.ds(ebase0 + ci * CHUNK, CHUNK)],
                                  webuf.at[b], sws[b]).wait()

            @pl.when(jnp.logical_not(first))
            def _():
                pltpu.make_async_copy(msg.at[b],
                                      acc_sh.at[dstv.at[islot(ci), 0]],
                                      sss[b]).wait()

            def edge_body(e, c2):
                for j in range(D // 16):
                    sl = pl.ds(j * 16, 16)
                    msg[b, e, sl] = rows[b, e, sl] * webuf[b, e, sl]
                return c2

            lax.fori_loop(0, CHUNK, edge_body, 0)
            pltpu.async_copy(msg.at[b], acc_sh.at[dstv.at[islot(ci), 0]],
                             sss[b], add=True)

            @pl.when(ci + 2 < NCHUNK)
            def _():
                issue(ci + 2, b)

        issue(0, 0)
        issue(1, 1)

        def group_body(g, carry):
            gb = g * GRP
            step(gb + 0, 0, g == 0)
            step(gb + 1, 1, g == 0)
            @pl.when(jnp.logical_and(g >= 1, g + 2 < NGRP))
            def _():
                idx_issue(g + 2)

            def pair_body(k, c2):
                step(gb + 2 * k, 0, False)
                step(gb + 2 * k + 1, 1, False)
                return c2

            lax.fori_loop(1, 4, pair_body, 0)
            @pl.when(jnp.logical_and(g + 1 >= IRING, g + 1 < NGRP))
            def _():
                idx_wait(g + 1)
            step(gb + 8, 0, False)
            step(gb + 9, 1, False)
            return carry

        lax.fori_loop(0, NGRP, group_body, 0)
        pltpu.make_async_copy(msg.at[0], acc_sh.at[dstv.at[0, 0]],
                              sss[0]).wait()
        pltpu.make_async_copy(msg.at[1], acc_sh.at[dstv.at[1, 0]],
                              sss[1]).wait()
        plsc.subcore_barrier()
        pltpu.sync_copy(acc_sh.at[pl.ds(r0, RPW)],
                        out_hbm.at[cid, pl.ds(r0, RPW)])

    return sc_agg


@functools.cache
def _sc_agg_cached():
    return _sc_agg_build()


def _sc_agg_call(x, we, src, dst, zeros_rpw):
    return _sc_agg_cached()(x, we, src, dst, zeros_rpw)


def _finish_body(p_ref, sc_ref, w2_ref, g_ref, brow_ref, out_ref):
    agg = (p_ref[0] + p_ref[1]) * CN
    y = jnp.dot(agg, w2_ref[...], preferred_element_type=jnp.float32) + sc_ref[...]
    n2 = jnp.dot(y * y, g_ref[...], preferred_element_type=jnp.float32) + 1e-8
    nrm = jnp.sqrt(n2)
    t = nrm + brow_ref[...]
    sp = jnp.maximum(t, 0.0) + jnp.log1p(jnp.exp(-jnp.abs(t))) - LOG2
    out_ref[...] = y * (sp / nrm)


def _finish(parts, sc, w2, g, brow):
    return pl.pallas_call(
        _finish_body,
        grid=(N // BN,),
        in_specs=[
            pl.BlockSpec((NSC, BN, D), lambda i: (0, i, 0)),
            pl.BlockSpec((BN, D), lambda i: (i, 0)),
            pl.BlockSpec((D, D), lambda i: (0, 0)),
            pl.BlockSpec((D, D), lambda i: (0, 0)),
            pl.BlockSpec((1, D), lambda i: (0, 0)),
        ],
        out_specs=pl.BlockSpec((BN, D), lambda i: (i, 0)),
        out_shape=jax.ShapeDtypeStruct((N, D), jnp.float32),
    )(parts, sc, w2, g, brow)


def _norm_groups():
    p = jnp.arange(D)[:, None]
    q = jnp.arange(D)[None, :]
    scal = (p == q) & (q < NS)
    vec = (p >= NS) & (q >= NS) & ((p - NS) // 3 == (q - NS) // 3)
    return (scal | vec).astype(jnp.float32)


def _layer(feat, attrs, src, dst, ee, ea, w1, wfc1, wfc2, wem, w2, wsc, bias,
           zeros_rpw, g):
    x, sc = _node_prep(feat, attrs, w1, wsc)
    we = _edge_we(ee, ea, wfc1, wfc2, wem)
    parts = _sc_agg_call(x, we, src, dst, zeros_rpw)
    brow = jnp.where(jnp.arange(D) < NS, bias[0], bias[1])[None, :]
    return _finish(parts, sc, w2, g, brow)


def kernel(node_features, node_attrs, edge_index, edge_embedding, edge_attrs,
           W1_0, Wfc1_0, Wfc2_0, We_0, W2_0, Wsc_0, bias_0,
           W1_1, Wfc1_1, Wfc2_1, We_1, W2_1, Wsc_1, bias_1):
    src = edge_index[0].astype(jnp.int32).reshape(E // CHUNK, 1, CHUNK)
    dst = edge_index[1].astype(jnp.int32).reshape(E // CHUNK, 1, CHUNK)
    zeros_rpw = jnp.zeros((RPW, D), dtype=jnp.float32)
    g = _norm_groups()
    h = _layer(node_features, node_attrs, src, dst, edge_embedding, edge_attrs,
               W1_0, Wfc1_0, Wfc2_0, We_0, W2_0, Wsc_0, bias_0, zeros_rpw, g)
    h = _layer(h, node_attrs, src, dst, edge_embedding, edge_attrs,
               W1_1, Wfc1_1, Wfc2_1, We_1, W2_1, Wsc_1, bias_1, zeros_rpw, g)
    return h
